# Initial kernel scaffold; baseline (speedup 1.0000x reference)
import jax
import jax.numpy as jnp
from jax.experimental import pallas as pl


def kernel(x, edge_index, c0_Wq, c0_bq, c0_Wk, c0_bk, c0_Wv, c0_bv, c0_Ws, c0_bs, c0_Wbeta, c1_Wq, c1_bq, c1_Wk, c1_bk, c1_Wv, c1_bv, c1_Ws, c1_bs, c1_Wbeta, ln0_w, ln0_b, ln1_w, ln1_b, l1_W, l1_b, l2_W, l2_b):
    raise NotImplementedError("write your pallas kernel here")


# speedup vs baseline: 13.8367x; 13.8367x over previous
import functools
import math

import jax
import jax.numpy as jnp
from jax import lax
from jax.experimental import pallas as pl
from jax.experimental.pallas import tpu as pltpu
from jax.experimental.pallas import tpu_sc as plsc

HID = 128
HEADS = 4
DH = HID // HEADS
OUT_DIM = 64
ACCW = HID + 16


def _proj_body(x_ref, w_ref, b_ref, q_ref, kv_ref, r_ref):
    y = jnp.dot(x_ref[...], w_ref[...], preferred_element_type=jnp.float32)
    y = y + b_ref[...]
    q_ref[...] = y[:, 0:HID]
    kv_ref[...] = y[:, HID:3 * HID]
    r_ref[...] = y[:, 3 * HID:4 * HID]


def _proj(x, W, b, BN):
    n = x.shape[0]
    grid = n // BN
    return pl.pallas_call(
        _proj_body,
        grid=(grid,),
        in_specs=[
            pl.BlockSpec((BN, x.shape[1]), lambda i: (i, 0)),
            pl.BlockSpec((x.shape[1], 4 * HID), lambda i: (0, 0)),
            pl.BlockSpec((1, 4 * HID), lambda i: (0, 0)),
        ],
        out_specs=[
            pl.BlockSpec((BN, HID), lambda i: (i, 0)),
            pl.BlockSpec((BN, 2 * HID), lambda i: (i, 0)),
            pl.BlockSpec((BN, HID), lambda i: (i, 0)),
        ],
        out_shape=[
            jax.ShapeDtypeStruct((n, HID), jnp.float32),
            jax.ShapeDtypeStruct((n, 2 * HID), jnp.float32),
            jax.ShapeDtypeStruct((n, HID), jnp.float32),
        ],
    )(x, W, b)


def _merge_core(accA, accB, r, s4, wbo, wbr, wbd, lnw, lnb):
    acc = accA + accB
    den4 = acc[:, HID:HID + HEADS]
    den = jnp.dot(den4, s4, preferred_element_type=jnp.float32)
    out = acc[:, 0:HID] / (den + 1e-16)
    logit = jnp.sum(out * wbo + r * wbr + (out - r) * wbd, axis=1, keepdims=True)
    beta = jax.nn.sigmoid(logit)
    h = beta * r + (1.0 - beta) * out
    mu = jnp.mean(h, axis=1, keepdims=True)
    hc = h - mu
    var = jnp.mean(hc * hc, axis=1, keepdims=True)
    h = hc * jax.lax.rsqrt(var + 1e-5) * lnw + lnb
    return jnp.maximum(h, 0.0)


def _mid_body(accA_ref, accB_ref, r_ref, s4_ref, wb_ref, ln_ref, w_ref, b_ref,
              q_ref, kv_ref, r1_ref, hres_ref):
    h = _merge_core(accA_ref[...], accB_ref[...], r_ref[...], s4_ref[...],
                    wb_ref[0:1, :], wb_ref[1:2, :], wb_ref[2:3, :],
                    ln_ref[0:1, :], ln_ref[1:2, :])
    hres_ref[...] = h
    y = jnp.dot(h, w_ref[...], preferred_element_type=jnp.float32) + b_ref[...]
    q_ref[...] = y[:, 0:HID]
    kv_ref[...] = y[:, HID:3 * HID]
    r1_ref[...] = y[:, 3 * HID:4 * HID]


def _mid(accA, accB, r, s4, wb, ln, W, b, BN):
    n = r.shape[0]
    grid = n // BN
    return pl.pallas_call(
        _mid_body,
        grid=(grid,),
        in_specs=[
            pl.BlockSpec((BN, ACCW), lambda i: (i, 0)),
            pl.BlockSpec((BN, ACCW), lambda i: (i, 0)),
            pl.BlockSpec((BN, HID), lambda i: (i, 0)),
            pl.BlockSpec((HEADS, HID), lambda i: (0, 0)),
            pl.BlockSpec((3, HID), lambda i: (0, 0)),
            pl.BlockSpec((2, HID), lambda i: (0, 0)),
            pl.BlockSpec((HID, 4 * HID), lambda i: (0, 0)),
            pl.BlockSpec((1, 4 * HID), lambda i: (0, 0)),
        ],
        out_specs=[
            pl.BlockSpec((BN, HID), lambda i: (i, 0)),
            pl.BlockSpec((BN, 2 * HID), lambda i: (i, 0)),
            pl.BlockSpec((BN, HID), lambda i: (i, 0)),
            pl.BlockSpec((BN, HID), lambda i: (i, 0)),
        ],
        out_shape=[
            jax.ShapeDtypeStruct((n, HID), jnp.float32),
            jax.ShapeDtypeStruct((n, 2 * HID), jnp.float32),
            jax.ShapeDtypeStruct((n, HID), jnp.float32),
            jax.ShapeDtypeStruct((n, HID), jnp.float32),
        ],
    )(accA, accB, r, s4, wb, ln, W, b)


def _fin_body(accA_ref, accB_ref, r_ref, s4_ref, wb_ref, ln_ref, hres_ref,
              w1_ref, b1_ref, w2_ref, b2_ref, o_ref):
    h = _merge_core(accA_ref[...], accB_ref[...], r_ref[...], s4_ref[...],
                    wb_ref[0:1, :], wb_ref[1:2, :], wb_ref[2:3, :],
                    ln_ref[0:1, :], ln_ref[1:2, :])
    h = h + hres_ref[...]
    h = jnp.dot(h, w1_ref[...], preferred_element_type=jnp.float32) + b1_ref[...]
    h = jnp.maximum(h, 0.0)
    o_ref[...] = jnp.dot(h, w2_ref[...], preferred_element_type=jnp.float32) + b2_ref[...]


def _fin(accA, accB, r, s4, wb, ln, hres, W1, b1, W2, b2, BN):
    n = r.shape[0]
    grid = n // BN
    return pl.pallas_call(
        _fin_body,
        grid=(grid,),
        in_specs=[
            pl.BlockSpec((BN, ACCW), lambda i: (i, 0)),
            pl.BlockSpec((BN, ACCW), lambda i: (i, 0)),
            pl.BlockSpec((BN, HID), lambda i: (i, 0)),
            pl.BlockSpec((HEADS, HID), lambda i: (0, 0)),
            pl.BlockSpec((3, HID), lambda i: (0, 0)),
            pl.BlockSpec((2, HID), lambda i: (0, 0)),
            pl.BlockSpec((BN, HID), lambda i: (i, 0)),
            pl.BlockSpec((HID, 2 * HID), lambda i: (0, 0)),
            pl.BlockSpec((1, 2 * HID), lambda i: (0, 0)),
            pl.BlockSpec((2 * HID, OUT_DIM), lambda i: (0, 0)),
            pl.BlockSpec((1, OUT_DIM), lambda i: (0, 0)),
        ],
        out_specs=[pl.BlockSpec((BN, OUT_DIM), lambda i: (i, 0))],
        out_shape=[jax.ShapeDtypeStruct((n, OUT_DIM), jnp.float32)],
    )(accA, accB, r, s4, wb, ln, hres, W1, b1, W2, b2)[0]


def _make_edge_kernel(n_nodes, n_edges, chunk):
    NW = 32
    epw = n_edges // NW
    nchunk = epw // chunk
    zrows = chunk
    nzc = n_nodes // zrows
    nzc_per = (nzc + 15) // 16
    inv_sqrt = 1.0 / math.sqrt(float(DH))

    mesh = plsc.VectorSubcoreMesh(core_axis_name="c", subcore_axis_name="s",
                                  num_cores=2, num_subcores=16)

    @functools.partial(
        pl.kernel,
        out_type=jax.ShapeDtypeStruct((2 * n_nodes, ACCW), jnp.float32),
        mesh=mesh,
        scratch_types=[
            pltpu.VMEM((chunk,), jnp.int32),
            pltpu.VMEM((chunk,), jnp.int32),
            pltpu.VMEM((chunk, HID), jnp.float32),
            pltpu.VMEM((chunk, 2 * HID), jnp.float32),
            pltpu.VMEM((chunk, ACCW), jnp.float32),
            pltpu.VMEM_SHARED((n_nodes, ACCW), jnp.float32),
            pltpu.SemaphoreType.DMA,
            pltpu.SemaphoreType.DMA,
        ],
        compiler_params=pltpu.CompilerParams(use_tc_tiling_on_sc=False),
    )
    def edge_kernel(q_hbm, kv_hbm, src_hbm, dst_hbm, out_hbm,
                    srcv, dstv, qr, kvr, contrib, acc, sem1, sem2):
        cid = lax.axis_index("c")
        sid = lax.axis_index("s")

        def zero_row(i, carry):
            for j in range(ACCW // 16):
                contrib[i, pl.ds(j * 16, 16)] = jnp.zeros((16,), jnp.float32)
            return carry

        lax.fori_loop(0, zrows, zero_row, 0)

        def zero_copy(zc, carry):
            g = sid + zc * 16

            @pl.when(g < nzc)
            def _():
                pltpu.sync_copy(contrib, acc.at[pl.ds(g * zrows, zrows)])

            return carry

        lax.fori_loop(0, nzc_per, zero_copy, 0)
        plsc.subcore_barrier()

        lane = lax.iota(jnp.int32, 16)
        xor_perms = [jnp.bitwise_xor(lane, k) for k in (1, 2, 4, 8)]
        base0 = (cid * 16 + sid) * epw

        def chunk_body(ci, carry):
            base = base0 + ci * chunk
            pltpu.sync_copy(src_hbm.at[pl.ds(base, chunk)], srcv)
            pltpu.sync_copy(dst_hbm.at[pl.ds(base, chunk)], dstv)
            pltpu.sync_copy(kv_hbm.at[srcv], kvr)
            pltpu.sync_copy(q_hbm.at[dstv], qr)

            def edge_body(i, carry2):
                den = jnp.zeros((16,), jnp.float32)
                for h in range(HEADS):
                    a = (qr[i, pl.ds(h * DH, 16)] * kvr[i, pl.ds(h * DH, 16)]
                         + qr[i, pl.ds(h * DH + 16, 16)] * kvr[i, pl.ds(h * DH + 16, 16)])
                    for p in xor_perms:
                        a = a + a.at[p].get(mode="promise_in_bounds",
                                            unique_indices=True)
                    ev = jnp.exp(a * inv_sqrt)
                    contrib[i, pl.ds(h * DH, 16)] = kvr[i, pl.ds(HID + h * DH, 16)] * ev
                    contrib[i, pl.ds(h * DH + 16, 16)] = kvr[i, pl.ds(HID + h * DH + 16, 16)] * ev
                    den = jnp.where(lane == h, ev, den)
                contrib[i, pl.ds(HID, 16)] = den
                return carry2

            lax.fori_loop(0, chunk, edge_body, 0)
            pltpu.sync_copy(contrib, acc.at[dstv], add=True)
            return carry

        lax.fori_loop(0, nchunk, chunk_body, 0)
        plsc.subcore_barrier()

        def dump(zc, carry):
            g = sid + zc * 16

            @pl.when(g < nzc)
            def _():
                r0 = g * zrows
                pltpu.sync_copy(acc.at[pl.ds(r0, zrows)],
                                out_hbm.at[pl.ds(cid * n_nodes + r0, zrows)])

            return carry

        lax.fori_loop(0, nzc_per, dump, 0)

    return edge_kernel


def kernel(x, edge_index, c0_Wq, c0_bq, c0_Wk, c0_bk, c0_Wv, c0_bv, c0_Ws,
           c0_bs, c0_Wbeta, c1_Wq, c1_bq, c1_Wk, c1_bk, c1_Wv, c1_bv, c1_Ws,
           c1_bs, c1_Wbeta, ln0_w, ln0_b, ln1_w, ln1_b, l1_W, l1_b, l2_W, l2_b):
    n = x.shape[0]
    n_edges = edge_index.shape[1]
    BN = 1000
    CHUNK = 40

    src = edge_index[0].astype(jnp.int32)
    dst = edge_index[1].astype(jnp.int32)

    W0 = jnp.concatenate([c0_Wq, c0_Wk, c0_Wv, c0_Ws], axis=1)
    b0 = jnp.concatenate([c0_bq, c0_bk, c0_bv, c0_bs])[None, :]
    W1 = jnp.concatenate([c1_Wq, c1_Wk, c1_Wv, c1_Ws], axis=1)
    b1 = jnp.concatenate([c1_bq, c1_bk, c1_bv, c1_bs])[None, :]
    s4 = jnp.repeat(jnp.eye(HEADS, dtype=jnp.float32), DH, axis=1)
    wb0 = jnp.stack([c0_Wbeta[0:HID, 0], c0_Wbeta[HID:2 * HID, 0],
                     c0_Wbeta[2 * HID:3 * HID, 0]])
    wb1 = jnp.stack([c1_Wbeta[0:HID, 0], c1_Wbeta[HID:2 * HID, 0],
                     c1_Wbeta[2 * HID:3 * HID, 0]])
    ln0 = jnp.stack([ln0_w, ln0_b])
    ln1 = jnp.stack([ln1_w, ln1_b])

    edge_kernel = _make_edge_kernel(n, n_edges, CHUNK)

    q0, kv0, r0 = _proj(x, W0, b0, BN)
    acc0 = edge_kernel(q0, kv0, src, dst)
    q1, kv1, r1, hres = _mid(acc0[:n], acc0[n:], r0, s4, wb0, ln0, W1, b1, BN)
    acc1 = edge_kernel(q1, kv1, src, dst)
    out = _fin(acc1[:n], acc1[n:], r1, s4, wb1, ln1, hres,
               l1_W, l1_b[None, :], l2_W, l2_b[None, :], BN)
    return out.reshape(n, OUT_DIM, 1)


# speedup vs baseline: 20.4782x; 1.4800x over previous
import functools
import math

import jax
import jax.numpy as jnp
from jax import lax
from jax.experimental import pallas as pl
from jax.experimental.pallas import tpu as pltpu
from jax.experimental.pallas import tpu_sc as plsc

HID = 128
HEADS = 4
DH = HID // HEADS
OUT_DIM = 64
ACCW = HID + 16


def _proj_body(x_ref, w_ref, b_ref, q_ref, kv_ref, r_ref):
    y = jnp.dot(x_ref[...], w_ref[...], preferred_element_type=jnp.float32)
    y = y + b_ref[...]
    q_ref[...] = y[:, 0:HID]
    kv_ref[...] = y[:, HID:3 * HID]
    r_ref[...] = y[:, 3 * HID:4 * HID]


def _proj(x, W, b, BN):
    n = x.shape[0]
    grid = n // BN
    return pl.pallas_call(
        _proj_body,
        grid=(grid,),
        in_specs=[
            pl.BlockSpec((BN, x.shape[1]), lambda i: (i, 0)),
            pl.BlockSpec((x.shape[1], 4 * HID), lambda i: (0, 0)),
            pl.BlockSpec((1, 4 * HID), lambda i: (0, 0)),
        ],
        out_specs=[
            pl.BlockSpec((BN, HID), lambda i: (i, 0)),
            pl.BlockSpec((BN, 2 * HID), lambda i: (i, 0)),
            pl.BlockSpec((BN, HID), lambda i: (i, 0)),
        ],
        out_shape=[
            jax.ShapeDtypeStruct((n, HID), jnp.float32),
            jax.ShapeDtypeStruct((n, 2 * HID), jnp.float32),
            jax.ShapeDtypeStruct((n, HID), jnp.float32),
        ],
    )(x, W, b)


def _merge_core(accA, accB, r, s4, wbo, wbr, wbd, lnw, lnb):
    acc = accA + accB
    den4 = acc[:, HID:HID + HEADS]
    den = jnp.dot(den4, s4, preferred_element_type=jnp.float32)
    out = acc[:, 0:HID] / (den + 1e-16)
    logit = jnp.sum(out * wbo + r * wbr + (out - r) * wbd, axis=1, keepdims=True)
    beta = jax.nn.sigmoid(logit)
    h = beta * r + (1.0 - beta) * out
    mu = jnp.mean(h, axis=1, keepdims=True)
    hc = h - mu
    var = jnp.mean(hc * hc, axis=1, keepdims=True)
    h = hc * jax.lax.rsqrt(var + 1e-5) * lnw + lnb
    return jnp.maximum(h, 0.0)


def _mid_body(accA_ref, accB_ref, r_ref, s4_ref, wb_ref, ln_ref, w_ref, b_ref,
              q_ref, kv_ref, r1_ref, hres_ref):
    h = _merge_core(accA_ref[...], accB_ref[...], r_ref[...], s4_ref[...],
                    wb_ref[0:1, :], wb_ref[1:2, :], wb_ref[2:3, :],
                    ln_ref[0:1, :], ln_ref[1:2, :])
    hres_ref[...] = h
    y = jnp.dot(h, w_ref[...], preferred_element_type=jnp.float32) + b_ref[...]
    q_ref[...] = y[:, 0:HID]
    kv_ref[...] = y[:, HID:3 * HID]
    r1_ref[...] = y[:, 3 * HID:4 * HID]


def _mid(accA, accB, r, s4, wb, ln, W, b, BN):
    n = r.shape[0]
    grid = n // BN
    return pl.pallas_call(
        _mid_body,
        grid=(grid,),
        in_specs=[
            pl.BlockSpec((BN, ACCW), lambda i: (i, 0)),
            pl.BlockSpec((BN, ACCW), lambda i: (i, 0)),
            pl.BlockSpec((BN, HID), lambda i: (i, 0)),
            pl.BlockSpec((HEADS, HID), lambda i: (0, 0)),
            pl.BlockSpec((3, HID), lambda i: (0, 0)),
            pl.BlockSpec((2, HID), lambda i: (0, 0)),
            pl.BlockSpec((HID, 4 * HID), lambda i: (0, 0)),
            pl.BlockSpec((1, 4 * HID), lambda i: (0, 0)),
        ],
        out_specs=[
            pl.BlockSpec((BN, HID), lambda i: (i, 0)),
            pl.BlockSpec((BN, 2 * HID), lambda i: (i, 0)),
            pl.BlockSpec((BN, HID), lambda i: (i, 0)),
            pl.BlockSpec((BN, HID), lambda i: (i, 0)),
        ],
        out_shape=[
            jax.ShapeDtypeStruct((n, HID), jnp.float32),
            jax.ShapeDtypeStruct((n, 2 * HID), jnp.float32),
            jax.ShapeDtypeStruct((n, HID), jnp.float32),
            jax.ShapeDtypeStruct((n, HID), jnp.float32),
        ],
    )(accA, accB, r, s4, wb, ln, W, b)


def _fin_body(accA_ref, accB_ref, r_ref, s4_ref, wb_ref, ln_ref, hres_ref,
              w1_ref, b1_ref, w2_ref, b2_ref, o_ref):
    h = _merge_core(accA_ref[...], accB_ref[...], r_ref[...], s4_ref[...],
                    wb_ref[0:1, :], wb_ref[1:2, :], wb_ref[2:3, :],
                    ln_ref[0:1, :], ln_ref[1:2, :])
    h = h + hres_ref[...]
    h = jnp.dot(h, w1_ref[...], preferred_element_type=jnp.float32) + b1_ref[...]
    h = jnp.maximum(h, 0.0)
    o_ref[...] = jnp.dot(h, w2_ref[...], preferred_element_type=jnp.float32) + b2_ref[...]


def _fin(accA, accB, r, s4, wb, ln, hres, W1, b1, W2, b2, BN):
    n = r.shape[0]
    grid = n // BN
    return pl.pallas_call(
        _fin_body,
        grid=(grid,),
        in_specs=[
            pl.BlockSpec((BN, ACCW), lambda i: (i, 0)),
            pl.BlockSpec((BN, ACCW), lambda i: (i, 0)),
            pl.BlockSpec((BN, HID), lambda i: (i, 0)),
            pl.BlockSpec((HEADS, HID), lambda i: (0, 0)),
            pl.BlockSpec((3, HID), lambda i: (0, 0)),
            pl.BlockSpec((2, HID), lambda i: (0, 0)),
            pl.BlockSpec((BN, HID), lambda i: (i, 0)),
            pl.BlockSpec((HID, 2 * HID), lambda i: (0, 0)),
            pl.BlockSpec((1, 2 * HID), lambda i: (0, 0)),
            pl.BlockSpec((2 * HID, OUT_DIM), lambda i: (0, 0)),
            pl.BlockSpec((1, OUT_DIM), lambda i: (0, 0)),
        ],
        out_specs=[pl.BlockSpec((BN, OUT_DIM), lambda i: (i, 0))],
        out_shape=[jax.ShapeDtypeStruct((n, OUT_DIM), jnp.float32)],
    )(accA, accB, r, s4, wb, ln, hres, W1, b1, W2, b2)[0]


def _make_edge_kernel(n_nodes, n_edges, chunk):
    NW = 32
    epw = n_edges // NW
    nchunk = epw // chunk
    zrows = chunk
    nzc = n_nodes // zrows
    nzc_per = (nzc + 15) // 16
    inv_sqrt = 1.0 / math.sqrt(float(DH))

    mesh = plsc.VectorSubcoreMesh(core_axis_name="c", subcore_axis_name="s",
                                  num_cores=2, num_subcores=16)

    @functools.partial(
        pl.kernel,
        out_type=jax.ShapeDtypeStruct((2 * n_nodes, ACCW), jnp.float32),
        mesh=mesh,
        scratch_types=[
            [pltpu.VMEM((chunk,), jnp.int32)] * 2,
            [pltpu.VMEM((chunk,), jnp.int32)] * 2,
            [pltpu.VMEM((chunk, HID), jnp.float32)] * 2,
            [pltpu.VMEM((chunk, 2 * HID), jnp.float32)] * 2,
            pltpu.VMEM((chunk, ACCW), jnp.float32),
            pltpu.VMEM((chunk,), jnp.int32),
            pltpu.VMEM_SHARED((n_nodes, ACCW), jnp.float32),
            [pltpu.SemaphoreType.DMA] * 2,
            [pltpu.SemaphoreType.DMA] * 2,
            [pltpu.SemaphoreType.DMA] * 2,
            [pltpu.SemaphoreType.DMA] * 2,
        ],
        compiler_params=pltpu.CompilerParams(use_tc_tiling_on_sc=False),
    )
    def edge_kernel(q_hbm, kv_hbm, src_hbm, dst_hbm, out_hbm,
                    srcv, dstv, qr, kvr, contrib, dsts, acc,
                    semsi, semdi, semq, semk):
        cid = lax.axis_index("c")
        sid = lax.axis_index("s")

        def zero_row(i, carry):
            for j in range(ACCW // 16):
                contrib[i, pl.ds(j * 16, 16)] = jnp.zeros((16,), jnp.float32)
            return carry

        lax.fori_loop(0, zrows, zero_row, 0)

        def zero_copy(zc, carry):
            g = sid + zc * 16

            @pl.when(g < nzc)
            def _():
                pltpu.sync_copy(contrib, acc.at[pl.ds(g * zrows, zrows)])

            return carry

        lax.fori_loop(0, nzc_per, zero_copy, 0)
        plsc.subcore_barrier()

        lane = lax.iota(jnp.int32, 16)
        xor_perms = [jnp.bitwise_xor(lane, k) for k in (1, 2, 4, 8)]
        base0 = (cid * 16 + sid) * epw

        def issue_idx(ci, b):
            base = base0 + ci * chunk
            pltpu.async_copy(src_hbm.at[pl.ds(base, chunk)], srcv[b], semsi[b])
            pltpu.async_copy(dst_hbm.at[pl.ds(base, chunk)], dstv[b], semdi[b])

        def wait_idx(ci, b):
            base = base0 + ci * chunk
            pltpu.make_async_copy(src_hbm.at[pl.ds(base, chunk)], srcv[b],
                                  semsi[b]).wait()
            pltpu.make_async_copy(dst_hbm.at[pl.ds(base, chunk)], dstv[b],
                                  semdi[b]).wait()

        def issue_gather(b):
            pltpu.async_copy(kv_hbm.at[srcv[b]], kvr[b], semk[b])
            pltpu.async_copy(q_hbm.at[dstv[b]], qr[b], semq[b])

        def wait_gather(b):
            pltpu.make_async_copy(kv_hbm.at[srcv[b]], kvr[b], semk[b]).wait()
            pltpu.make_async_copy(q_hbm.at[dstv[b]], qr[b], semq[b]).wait()

        def compute(b):
            qrb = qr[b]
            kvrb = kvr[b]

            def edge_body(i, carry2):
                den = jnp.zeros((16,), jnp.float32)
                for h in range(HEADS):
                    a = (qrb[i, pl.ds(h * DH, 16)] * kvrb[i, pl.ds(h * DH, 16)]
                         + qrb[i, pl.ds(h * DH + 16, 16)] * kvrb[i, pl.ds(h * DH + 16, 16)])
                    for p in xor_perms:
                        a = a + a.at[p].get(mode="promise_in_bounds",
                                            unique_indices=True)
                    ev = jnp.exp(a * inv_sqrt)
                    contrib[i, pl.ds(h * DH, 16)] = kvrb[i, pl.ds(HID + h * DH, 16)] * ev
                    contrib[i, pl.ds(h * DH + 16, 16)] = kvrb[i, pl.ds(HID + h * DH + 16, 16)] * ev
                    den = jnp.where(lane == h, ev, den)
                contrib[i, pl.ds(HID, 16)] = den
                return carry2

            lax.fori_loop(0, chunk, edge_body, 0)
            pltpu.sync_copy(contrib, acc.at[dsts], add=True)

        snap_offs = list(range(0, chunk - 15, 16))
        if snap_offs[-1] != chunk - 16:
            snap_offs.append(chunk - 16)

        def snap_dst(b):
            for off in snap_offs:
                dsts[pl.ds(off, 16)] = dstv[b][pl.ds(off, 16)]

        issue_idx(0, 0)
        issue_idx(1, 1)
        wait_idx(0, 0)
        issue_gather(0)

        def pipe_body(it, carry):
            c0 = 2 * it
            wait_gather(0)
            snap_dst(0)

            @pl.when(c0 + 2 < nchunk)
            def _():
                issue_idx(c0 + 2, 0)

            wait_idx(c0 + 1, 1)
            issue_gather(1)
            compute(0)
            wait_gather(1)
            snap_dst(1)

            @pl.when(c0 + 3 < nchunk)
            def _():
                issue_idx(c0 + 3, 1)

            @pl.when(c0 + 2 < nchunk)
            def _():
                wait_idx(c0 + 2, 0)
                issue_gather(0)

            compute(1)
            return carry

        lax.fori_loop(0, nchunk // 2, pipe_body, 0)
        plsc.subcore_barrier()

        def dump(zc, carry):
            g = sid + zc * 16

            @pl.when(g < nzc)
            def _():
                r0 = g * zrows
                pltpu.sync_copy(acc.at[pl.ds(r0, zrows)],
                                out_hbm.at[pl.ds(cid * n_nodes + r0, zrows)])

            return carry

        lax.fori_loop(0, nzc_per, dump, 0)

    return edge_kernel


def kernel(x, edge_index, c0_Wq, c0_bq, c0_Wk, c0_bk, c0_Wv, c0_bv, c0_Ws,
           c0_bs, c0_Wbeta, c1_Wq, c1_bq, c1_Wk, c1_bk, c1_Wv, c1_bv, c1_Ws,
           c1_bs, c1_Wbeta, ln0_w, ln0_b, ln1_w, ln1_b, l1_W, l1_b, l2_W, l2_b):
    n = x.shape[0]
    n_edges = edge_index.shape[1]
    BN = 1000
    CHUNK = 40

    src = edge_index[0].astype(jnp.int32)
    dst = edge_index[1].astype(jnp.int32)

    W0 = jnp.concatenate([c0_Wq, c0_Wk, c0_Wv, c0_Ws], axis=1)
    b0 = jnp.concatenate([c0_bq, c0_bk, c0_bv, c0_bs])[None, :]
    W1 = jnp.concatenate([c1_Wq, c1_Wk, c1_Wv, c1_Ws], axis=1)
    b1 = jnp.concatenate([c1_bq, c1_bk, c1_bv, c1_bs])[None, :]
    s4 = jnp.repeat(jnp.eye(HEADS, dtype=jnp.float32), DH, axis=1)
    wb0 = jnp.stack([c0_Wbeta[0:HID, 0], c0_Wbeta[HID:2 * HID, 0],
                     c0_Wbeta[2 * HID:3 * HID, 0]])
    wb1 = jnp.stack([c1_Wbeta[0:HID, 0], c1_Wbeta[HID:2 * HID, 0],
                     c1_Wbeta[2 * HID:3 * HID, 0]])
    ln0 = jnp.stack([ln0_w, ln0_b])
    ln1 = jnp.stack([ln1_w, ln1_b])

    edge_kernel = _make_edge_kernel(n, n_edges, CHUNK)

    q0, kv0, r0 = _proj(x, W0, b0, BN)
    acc0 = edge_kernel(q0, kv0, src, dst)
    q1, kv1, r1, hres = _mid(acc0[:n], acc0[n:], r0, s4, wb0, ln0, W1, b1, BN)
    acc1 = edge_kernel(q1, kv1, src, dst)
    out = _fin(acc1[:n], acc1[n:], r1, s4, wb1, ln1, hres,
               l1_W, l1_b[None, :], l2_W, l2_b[None, :], BN)
    return out.reshape(n, OUT_DIM, 1)


# speedup vs baseline: 20.5585x; 1.0039x over previous
import functools
import math

import jax
import jax.numpy as jnp
from jax import lax
from jax.experimental import pallas as pl
from jax.experimental.pallas import tpu as pltpu
from jax.experimental.pallas import tpu_sc as plsc

HID = 128
HEADS = 4
DH = HID // HEADS
OUT_DIM = 64
ACCW = HID + 16


def _proj_body(x_ref, w_ref, b_ref, q_ref, kv_ref, r_ref):
    y = jnp.dot(x_ref[...], w_ref[...], preferred_element_type=jnp.float32)
    y = y + b_ref[...]
    q_ref[...] = y[:, 0:HID]
    kv_ref[...] = y[:, HID:3 * HID]
    r_ref[...] = y[:, 3 * HID:4 * HID]


def _proj(x, W, b, BN):
    n = x.shape[0]
    grid = n // BN
    return pl.pallas_call(
        _proj_body,
        grid=(grid,),
        in_specs=[
            pl.BlockSpec((BN, x.shape[1]), lambda i: (i, 0)),
            pl.BlockSpec((x.shape[1], 4 * HID), lambda i: (0, 0)),
            pl.BlockSpec((1, 4 * HID), lambda i: (0, 0)),
        ],
        out_specs=[
            pl.BlockSpec((BN, HID), lambda i: (i, 0)),
            pl.BlockSpec((BN, 2 * HID), lambda i: (i, 0)),
            pl.BlockSpec((BN, HID), lambda i: (i, 0)),
        ],
        out_shape=[
            jax.ShapeDtypeStruct((n, HID), jnp.float32),
            jax.ShapeDtypeStruct((n, 2 * HID), jnp.float32),
            jax.ShapeDtypeStruct((n, HID), jnp.float32),
        ],
    )(x, W, b)


def _merge_core(accA, accB, r, s4, wbo, wbr, wbd, lnw, lnb):
    acc = accA + accB
    den4 = acc[:, HID:HID + HEADS]
    den = jnp.dot(den4, s4, preferred_element_type=jnp.float32)
    out = acc[:, 0:HID] / (den + 1e-16)
    logit = jnp.sum(out * wbo + r * wbr + (out - r) * wbd, axis=1, keepdims=True)
    beta = jax.nn.sigmoid(logit)
    h = beta * r + (1.0 - beta) * out
    mu = jnp.mean(h, axis=1, keepdims=True)
    hc = h - mu
    var = jnp.mean(hc * hc, axis=1, keepdims=True)
    h = hc * jax.lax.rsqrt(var + 1e-5) * lnw + lnb
    return jnp.maximum(h, 0.0)


def _mid_body(accA_ref, accB_ref, r_ref, s4_ref, wb_ref, ln_ref, w_ref, b_ref,
              q_ref, kv_ref, r1_ref, hres_ref):
    h = _merge_core(accA_ref[...], accB_ref[...], r_ref[...], s4_ref[...],
                    wb_ref[0:1, :], wb_ref[1:2, :], wb_ref[2:3, :],
                    ln_ref[0:1, :], ln_ref[1:2, :])
    hres_ref[...] = h
    y = jnp.dot(h, w_ref[...], preferred_element_type=jnp.float32) + b_ref[...]
    q_ref[...] = y[:, 0:HID]
    kv_ref[...] = y[:, HID:3 * HID]
    r1_ref[...] = y[:, 3 * HID:4 * HID]


def _mid(accA, accB, r, s4, wb, ln, W, b, BN):
    n = r.shape[0]
    grid = n // BN
    return pl.pallas_call(
        _mid_body,
        grid=(grid,),
        in_specs=[
            pl.BlockSpec((BN, ACCW), lambda i: (i, 0)),
            pl.BlockSpec((BN, ACCW), lambda i: (i, 0)),
            pl.BlockSpec((BN, HID), lambda i: (i, 0)),
            pl.BlockSpec((HEADS, HID), lambda i: (0, 0)),
            pl.BlockSpec((3, HID), lambda i: (0, 0)),
            pl.BlockSpec((2, HID), lambda i: (0, 0)),
            pl.BlockSpec((HID, 4 * HID), lambda i: (0, 0)),
            pl.BlockSpec((1, 4 * HID), lambda i: (0, 0)),
        ],
        out_specs=[
            pl.BlockSpec((BN, HID), lambda i: (i, 0)),
            pl.BlockSpec((BN, 2 * HID), lambda i: (i, 0)),
            pl.BlockSpec((BN, HID), lambda i: (i, 0)),
            pl.BlockSpec((BN, HID), lambda i: (i, 0)),
        ],
        out_shape=[
            jax.ShapeDtypeStruct((n, HID), jnp.float32),
            jax.ShapeDtypeStruct((n, 2 * HID), jnp.float32),
            jax.ShapeDtypeStruct((n, HID), jnp.float32),
            jax.ShapeDtypeStruct((n, HID), jnp.float32),
        ],
    )(accA, accB, r, s4, wb, ln, W, b)


def _fin_body(accA_ref, accB_ref, r_ref, s4_ref, wb_ref, ln_ref, hres_ref,
              w1_ref, b1_ref, w2_ref, b2_ref, o_ref):
    h = _merge_core(accA_ref[...], accB_ref[...], r_ref[...], s4_ref[...],
                    wb_ref[0:1, :], wb_ref[1:2, :], wb_ref[2:3, :],
                    ln_ref[0:1, :], ln_ref[1:2, :])
    h = h + hres_ref[...]
    h = jnp.dot(h, w1_ref[...], preferred_element_type=jnp.float32) + b1_ref[...]
    h = jnp.maximum(h, 0.0)
    o_ref[...] = jnp.dot(h, w2_ref[...], preferred_element_type=jnp.float32) + b2_ref[...]


def _fin(accA, accB, r, s4, wb, ln, hres, W1, b1, W2, b2, BN):
    n = r.shape[0]
    grid = n // BN
    return pl.pallas_call(
        _fin_body,
        grid=(grid,),
        in_specs=[
            pl.BlockSpec((BN, ACCW), lambda i: (i, 0)),
            pl.BlockSpec((BN, ACCW), lambda i: (i, 0)),
            pl.BlockSpec((BN, HID), lambda i: (i, 0)),
            pl.BlockSpec((HEADS, HID), lambda i: (0, 0)),
            pl.BlockSpec((3, HID), lambda i: (0, 0)),
            pl.BlockSpec((2, HID), lambda i: (0, 0)),
            pl.BlockSpec((BN, HID), lambda i: (i, 0)),
            pl.BlockSpec((HID, 2 * HID), lambda i: (0, 0)),
            pl.BlockSpec((1, 2 * HID), lambda i: (0, 0)),
            pl.BlockSpec((2 * HID, OUT_DIM), lambda i: (0, 0)),
            pl.BlockSpec((1, OUT_DIM), lambda i: (0, 0)),
        ],
        out_specs=[pl.BlockSpec((BN, OUT_DIM), lambda i: (i, 0))],
        out_shape=[jax.ShapeDtypeStruct((n, OUT_DIM), jnp.float32)],
    )(accA, accB, r, s4, wb, ln, hres, W1, b1, W2, b2)[0]


def _make_edge_kernel(n_nodes, n_edges, chunk):
    NW = 32
    epw = n_edges // NW
    nchunk = epw // chunk
    zrows = chunk
    nzc = n_nodes // zrows
    nzc_per = (nzc + 15) // 16
    inv_sqrt = 1.0 / math.sqrt(float(DH))

    mesh = plsc.VectorSubcoreMesh(core_axis_name="c", subcore_axis_name="s",
                                  num_cores=2, num_subcores=16)

    @functools.partial(
        pl.kernel,
        out_type=jax.ShapeDtypeStruct((2 * n_nodes, ACCW), jnp.float32),
        mesh=mesh,
        scratch_types=[
            [pltpu.VMEM((chunk,), jnp.int32)] * 2,
            [pltpu.VMEM((chunk,), jnp.int32)] * 2,
            [pltpu.VMEM((chunk, HID), jnp.float32)] * 2,
            [pltpu.VMEM((chunk, 2 * HID), jnp.float32)] * 2,
            pltpu.VMEM((chunk, ACCW), jnp.float32),
            pltpu.VMEM((chunk,), jnp.int32),
            pltpu.VMEM_SHARED((n_nodes, ACCW), jnp.float32),
            [pltpu.SemaphoreType.DMA] * 2,
            [pltpu.SemaphoreType.DMA] * 2,
            [pltpu.SemaphoreType.DMA] * 2,
            [pltpu.SemaphoreType.DMA] * 2,
        ],
        compiler_params=pltpu.CompilerParams(use_tc_tiling_on_sc=False),
    )
    def edge_kernel(q_hbm, kv_hbm, src_hbm, dst_hbm, out_hbm,
                    srcv, dstv, qr, kvr, contrib, dsts, acc,
                    semsi, semdi, semq, semk):
        cid = lax.axis_index("c")
        sid = lax.axis_index("s")

        def zero_row(i, carry):
            for j in range(ACCW // 16):
                contrib[i, pl.ds(j * 16, 16)] = jnp.zeros((16,), jnp.float32)
            return carry

        lax.fori_loop(0, zrows, zero_row, 0)

        def zero_copy(zc, carry):
            g = sid + zc * 16

            @pl.when(g < nzc)
            def _():
                pltpu.sync_copy(contrib, acc.at[pl.ds(g * zrows, zrows)])

            return carry

        lax.fori_loop(0, nzc_per, zero_copy, 0)
        plsc.subcore_barrier()

        lane = lax.iota(jnp.int32, 16)
        xor_perms = [jnp.bitwise_xor(lane, k) for k in (1, 2, 4, 8)]
        base0 = (cid * 16 + sid) * epw

        def issue_idx(ci, b):
            base = base0 + ci * chunk
            pltpu.async_copy(src_hbm.at[pl.ds(base, chunk)], srcv[b], semsi[b])
            pltpu.async_copy(dst_hbm.at[pl.ds(base, chunk)], dstv[b], semdi[b])

        def wait_idx(ci, b):
            base = base0 + ci * chunk
            pltpu.make_async_copy(src_hbm.at[pl.ds(base, chunk)], srcv[b],
                                  semsi[b]).wait()
            pltpu.make_async_copy(dst_hbm.at[pl.ds(base, chunk)], dstv[b],
                                  semdi[b]).wait()

        def issue_gather(b):
            pltpu.async_copy(kv_hbm.at[srcv[b]], kvr[b], semk[b])
            pltpu.async_copy(q_hbm.at[dstv[b]], qr[b], semq[b])

        def wait_gather(b):
            pltpu.make_async_copy(kv_hbm.at[srcv[b]], kvr[b], semk[b]).wait()
            pltpu.make_async_copy(q_hbm.at[dstv[b]], qr[b], semq[b]).wait()

        def compute(b):
            qrb = qr[b]
            kvrb = kvr[b]

            def edge_body(j, carry2):
                for u in range(2):
                    i = 2 * j + u
                    den = jnp.zeros((16,), jnp.float32)
                    for h in range(HEADS):
                        a = (qrb[i, pl.ds(h * DH, 16)] * kvrb[i, pl.ds(h * DH, 16)]
                             + qrb[i, pl.ds(h * DH + 16, 16)] * kvrb[i, pl.ds(h * DH + 16, 16)])
                        for p in xor_perms:
                            a = a + a.at[p].get(mode="promise_in_bounds",
                                                unique_indices=True)
                        ev = jnp.exp(a * inv_sqrt)
                        contrib[i, pl.ds(h * DH, 16)] = kvrb[i, pl.ds(HID + h * DH, 16)] * ev
                        contrib[i, pl.ds(h * DH + 16, 16)] = kvrb[i, pl.ds(HID + h * DH + 16, 16)] * ev
                        den = jnp.where(lane == h, ev, den)
                    contrib[i, pl.ds(HID, 16)] = den
                return carry2

            lax.fori_loop(0, chunk // 2, edge_body, 0)
            pltpu.sync_copy(contrib, acc.at[dsts], add=True)

        snap_offs = list(range(0, chunk - 15, 16))
        if snap_offs[-1] != chunk - 16:
            snap_offs.append(chunk - 16)

        def snap_dst(b):
            for off in snap_offs:
                dsts[pl.ds(off, 16)] = dstv[b][pl.ds(off, 16)]

        issue_idx(0, 0)
        issue_idx(1, 1)
        wait_idx(0, 0)
        issue_gather(0)

        def pipe_body(it, carry):
            c0 = 2 * it
            wait_gather(0)
            snap_dst(0)

            @pl.when(c0 + 2 < nchunk)
            def _():
                issue_idx(c0 + 2, 0)

            wait_idx(c0 + 1, 1)
            issue_gather(1)
            compute(0)
            wait_gather(1)
            snap_dst(1)

            @pl.when(c0 + 3 < nchunk)
            def _():
                issue_idx(c0 + 3, 1)

            @pl.when(c0 + 2 < nchunk)
            def _():
                wait_idx(c0 + 2, 0)
                issue_gather(0)

            compute(1)
            return carry

        lax.fori_loop(0, nchunk // 2, pipe_body, 0)
        plsc.subcore_barrier()

        def dump(zc, carry):
            g = sid + zc * 16

            @pl.when(g < nzc)
            def _():
                r0 = g * zrows
                pltpu.sync_copy(acc.at[pl.ds(r0, zrows)],
                                out_hbm.at[pl.ds(cid * n_nodes + r0, zrows)])

            return carry

        lax.fori_loop(0, nzc_per, dump, 0)

    return edge_kernel


def kernel(x, edge_index, c0_Wq, c0_bq, c0_Wk, c0_bk, c0_Wv, c0_bv, c0_Ws,
           c0_bs, c0_Wbeta, c1_Wq, c1_bq, c1_Wk, c1_bk, c1_Wv, c1_bv, c1_Ws,
           c1_bs, c1_Wbeta, ln0_w, ln0_b, ln1_w, ln1_b, l1_W, l1_b, l2_W, l2_b):
    n = x.shape[0]
    n_edges = edge_index.shape[1]
    BN = 1000
    CHUNK = 40

    src = edge_index[0].astype(jnp.int32)
    dst = edge_index[1].astype(jnp.int32)

    W0 = jnp.concatenate([c0_Wq, c0_Wk, c0_Wv, c0_Ws], axis=1)
    b0 = jnp.concatenate([c0_bq, c0_bk, c0_bv, c0_bs])[None, :]
    W1 = jnp.concatenate([c1_Wq, c1_Wk, c1_Wv, c1_Ws], axis=1)
    b1 = jnp.concatenate([c1_bq, c1_bk, c1_bv, c1_bs])[None, :]
    s4 = jnp.repeat(jnp.eye(HEADS, dtype=jnp.float32), DH, axis=1)
    wb0 = jnp.stack([c0_Wbeta[0:HID, 0], c0_Wbeta[HID:2 * HID, 0],
                     c0_Wbeta[2 * HID:3 * HID, 0]])
    wb1 = jnp.stack([c1_Wbeta[0:HID, 0], c1_Wbeta[HID:2 * HID, 0],
                     c1_Wbeta[2 * HID:3 * HID, 0]])
    ln0 = jnp.stack([ln0_w, ln0_b])
    ln1 = jnp.stack([ln1_w, ln1_b])

    edge_kernel = _make_edge_kernel(n, n_edges, CHUNK)

    q0, kv0, r0 = _proj(x, W0, b0, BN)
    acc0 = edge_kernel(q0, kv0, src, dst)
    q1, kv1, r1, hres = _mid(acc0[:n], acc0[n:], r0, s4, wb0, ln0, W1, b1, BN)
    acc1 = edge_kernel(q1, kv1, src, dst)
    out = _fin(acc1[:n], acc1[n:], r1, s4, wb1, ln1, hres,
               l1_W, l1_b[None, :], l2_W, l2_b[None, :], BN)
    return out.reshape(n, OUT_DIM, 1)
